# Initial kernel scaffold; baseline (speedup 1.0000x reference)
import jax
import jax.numpy as jnp
from jax.experimental import pallas as pl


def kernel(x, token_table, pos_table):
    raise NotImplementedError("write your pallas kernel here")


# speedup vs baseline: 7.6930x; 7.6930x over previous
import functools

import jax
import jax.numpy as jnp
from jax import lax
from jax.experimental import pallas as pl
from jax.experimental.pallas import tpu as pltpu
from jax.experimental.pallas import tpu_sc as plsc

MAXLEN = 500
EMBED_DIM = 64
BATCH = 4096

NC = 2
NS = 16
NW = NC * NS
TOKENS = BATCH * MAXLEN
TOK_PER_W = TOKENS // NW
CHUNK = 2 * MAXLEN
CHUNKS_PER_W = TOK_PER_W // CHUNK


def _make_sc_kernel():
  mesh = plsc.VectorSubcoreMesh(core_axis_name="c", subcore_axis_name="s")

  @functools.partial(
      pl.kernel,
      mesh=mesh,
      out_type=jax.ShapeDtypeStruct((TOKENS, EMBED_DIM), jnp.float32),
      compiler_params=pltpu.CompilerParams(use_tc_tiling_on_sc=False),
      scratch_types=[
          pltpu.VMEM((CHUNK,), jnp.int32),
          pltpu.VMEM((CHUNK, EMBED_DIM), jnp.float32),
          pltpu.VMEM((MAXLEN, EMBED_DIM), jnp.float32),
          pltpu.SemaphoreType.DMA,
      ],
  )
  def k(x_hbm, tab_hbm, pos_hbm, out_hbm, idx_v, rows_v, pos_v, sem):
    wid = lax.axis_index("s") * NC + lax.axis_index("c")
    base_w = wid * TOK_PER_W

    pltpu.sync_copy(pos_hbm, pos_v)

    def chunk_body(ci, carry):
      base = base_w + ci * CHUNK
      pltpu.sync_copy(x_hbm.at[pl.ds(base, CHUNK)], idx_v)
      pltpu.async_copy(tab_hbm.at[idx_v], rows_v, sem).wait()

      def add_body(l, carry2):
        for h in range(CHUNK // MAXLEN):
          t = h * MAXLEN + l
          for c in range(EMBED_DIM // 16):
            s = pl.ds(c * 16, 16)
            rows_v[t, s] = rows_v[t, s] + pos_v[l, s]
        return carry2

      lax.fori_loop(0, MAXLEN, add_body, 0)
      pltpu.sync_copy(rows_v, out_hbm.at[pl.ds(base, CHUNK)])
      return carry

    lax.fori_loop(0, CHUNKS_PER_W, chunk_body, 0)

  return k


_sc_kernel = _make_sc_kernel()


def kernel(x, token_table, pos_table):
  x_flat = x.reshape(TOKENS).astype(jnp.int32)
  out = _sc_kernel(x_flat, token_table, pos_table)
  return out.reshape(BATCH, MAXLEN, EMBED_DIM)
